# Initial kernel scaffold; baseline (speedup 1.0000x reference)
import jax
import jax.numpy as jnp
from jax.experimental import pallas as pl


def kernel(grid_graph_edge_index, grid_graph_edge_attr, ray_graph_edge_index, ray_graph_edge_attr, x, params):
    raise NotImplementedError("write your pallas kernel here")


# speedup vs baseline: 2.8740x; 2.8740x over previous
import functools

import jax
import jax.numpy as jnp
from jax import lax
from jax.experimental import pallas as pl
from jax.experimental.pallas import tpu as pltpu
from jax.experimental.pallas import tpu_sc as plsc

_N = 10000
_E = 160000
_HID = 32
_HID2 = 16
_EP = 163840
_NWIN = _EP // 128
_NACC = 10112
_BE = 2048
_BN = 2000

_f32 = jnp.float32


def _mlp(x, w1, b1, w2, b2, w3, b3, g, bt):
    h = jnp.maximum(jnp.dot(x, w1, preferred_element_type=_f32) + b1, 0.0)
    h = jnp.maximum(jnp.dot(h, w2, preferred_element_type=_f32) + b2, 0.0)
    y = jnp.dot(h, w3, preferred_element_type=_f32) + b3
    mu = jnp.mean(y, axis=-1, keepdims=True)
    var = jnp.mean((y - mu) ** 2, axis=-1, keepdims=True)
    return (y - mu) * lax.rsqrt(var + 1e-5) * g + bt


def _full_spec(a):
    nd = a.ndim
    return pl.BlockSpec(a.shape, lambda i, _nd=nd: (0,) * _nd)


def _mlp_rows_kernel(x_ref, w1, b1, w2, b2, w3, b3, g, bt, o_ref):
    o_ref[...] = _mlp(x_ref[...], w1[...], b1[...], w2[...], b2[...],
                      w3[...], b3[...], g[...], bt[...])


def _tc_mlp_rows(x, p, out_dim, block_rows):
    rows = x.shape[0]
    ws = (p['l1']['W'], p['l1']['b'], p['l2']['W'], p['l2']['b'],
          p['l3']['W'], p['l3']['b'], p['gamma'], p['beta'])
    return pl.pallas_call(
        _mlp_rows_kernel,
        grid=(rows // block_rows,),
        in_specs=[pl.BlockSpec((block_rows, x.shape[1]), lambda i: (i, 0))]
                 + [_full_spec(w) for w in ws],
        out_specs=pl.BlockSpec((block_rows, out_dim), lambda i: (i, 0)),
        out_shape=jax.ShapeDtypeStruct((rows, out_dim), _f32),
    )(x, *ws)


def _msg_kernel(hsg_ref, hsr_ref, ag_ref, ar_ref,
                wall_g, b2_g, wall_r, b2_r, rmat, smat,
                og_ref, or_ref):
    rm = rmat[...]
    sm = smat[...]

    def one(h_ref, a_ref, wall, b2):
        hs = h_ref[...]
        y = jnp.dot(hs, wall[...], preferred_element_type=_f32)
        a = jnp.dot(a_ref[...], rm, preferred_element_type=_f32)
        return (jnp.dot(y * a, sm, preferred_element_type=_f32)
                + jnp.dot(hs, b2[...], preferred_element_type=_f32))

    og_ref[...] = one(hsg_ref, ag_ref, wall_g, b2_g)
    or_ref[...] = one(hsr_ref, ar_ref, wall_r, b2_r)


def _tc_messages(hsg, hsr, attr_g, attr_r, wall_g, b2_g, wall_r, b2_r, rmat, smat):
    ws = (wall_g, b2_g, wall_r, b2_r, rmat, smat)
    return pl.pallas_call(
        _msg_kernel,
        grid=(_EP // _BE,),
        in_specs=[pl.BlockSpec((_BE, _HID), lambda i: (i, 0)),
                  pl.BlockSpec((_BE, _HID), lambda i: (i, 0)),
                  pl.BlockSpec((_BE, _HID2), lambda i: (i, 0)),
                  pl.BlockSpec((_BE, _HID2), lambda i: (i, 0))]
                 + [_full_spec(w) for w in ws],
        out_specs=[pl.BlockSpec((_BE, _HID2), lambda i: (i, 0)),
                   pl.BlockSpec((_BE, _HID2), lambda i: (i, 0))],
        out_shape=[jax.ShapeDtypeStruct((_EP, _HID2), _f32),
                   jax.ShapeDtypeStruct((_EP, _HID2), _f32)],
    )(hsg, hsr, attr_g, attr_r, *ws)


def _combine_kernel(sg0, sg1, sr0, sr1, cg0, cg1, cr0, cr1, h_ref,
                    root_g, bias_g, root_r, bias_r,
                    w1, b1, w2, b2, w3, b3, g, bt, o_ref):
    h = h_ref[...]
    cnt_g = jnp.maximum(cg0[...] + cg1[...], 1.0)
    cnt_r = jnp.maximum(cr0[...] + cr1[...], 1.0)
    xg = (sg0[...] + sg1[...]) / cnt_g \
        + jnp.dot(h, root_g[...], preferred_element_type=_f32) + bias_g[...]
    xr = (sr0[...] + sr1[...]) / cnt_r \
        + jnp.dot(h, root_r[...], preferred_element_type=_f32) + bias_r[...]
    cat = jnp.concatenate([xg, xr], axis=1)
    y = _mlp(cat, w1[...], b1[...], w2[...], b2[...], w3[...], b3[...],
             g[...], bt[...])
    o_ref[...] = y + h


def _tc_combine(sg, sr, cg, cr, h, root_g, bias_g, root_r, bias_r, prep):
    ws = (root_g, bias_g, root_r, bias_r,
          prep['l1']['W'], prep['l1']['b'], prep['l2']['W'], prep['l2']['b'],
          prep['l3']['W'], prep['l3']['b'], prep['gamma'], prep['beta'])
    part = pl.BlockSpec((_BN, _HID2), lambda i: (i, 0))
    return pl.pallas_call(
        _combine_kernel,
        grid=(_N // _BN,),
        in_specs=[part] * 8
                 + [pl.BlockSpec((_BN, _HID), lambda i: (i, 0))]
                 + [_full_spec(w) for w in ws],
        out_specs=pl.BlockSpec((_BN, _HID), lambda i: (i, 0)),
        out_shape=jax.ShapeDtypeStruct((_N, _HID), _f32),
    )(sg[0], sg[1], sr[0], sr[1], cg[0], cg[1], cr[0], cr[1], h, *ws)


def _sc_gather(table, idx2d):
    mesh = plsc.VectorSubcoreMesh(core_axis_name="c", subcore_axis_name="s")

    @functools.partial(
        pl.kernel, mesh=mesh,
        out_type=jax.ShapeDtypeStruct((_EP, _HID), _f32),
        compiler_params=pltpu.CompilerParams(use_tc_tiling_on_sc=False))
    def k(t_hbm, i_hbm, o_hbm):
        def body(i_vmem, o_vmem):
            pltpu.sync_copy(t_hbm.at[i_vmem.at[0]], o_vmem)

        pltpu.emit_pipeline(
            body,
            grid=(_NWIN,),
            in_specs=[pl.BlockSpec((1, 128), lambda i: (0, i))],
            out_specs=[pl.BlockSpec((128, _HID), lambda i: (i, 0))],
            core_axis_name=("c", "s"),
            dimension_semantics=(pltpu.PARALLEL,),
        )(i_hbm, o_hbm)

    return k(table, idx2d)


_GROUPS = 5
_GW = 8


def _sc_scatter_add(msg, idxw, zeros):
    mesh = plsc.VectorSubcoreMesh(core_axis_name="c", subcore_axis_name="s")

    @functools.partial(
        pl.kernel, mesh=mesh,
        out_type=jax.ShapeDtypeStruct((2, _N, _HID2), _f32),
        compiler_params=pltpu.CompilerParams(use_tc_tiling_on_sc=False),
        scratch_types=[pltpu.VMEM((_GW, 128), jnp.int32),
                       pltpu.VMEM((_GW * 128, _HID2), _f32),
                       pltpu.VMEM_SHARED((_NACC, _HID2), _f32)])
    def k(m_hbm, i_hbm, z_hbm, o_hbm, idx_v, msg_v, acc):
        c = lax.axis_index("c")
        s = lax.axis_index("s")
        zrows = _NACC // 16
        pltpu.sync_copy(z_hbm.at[pl.ds(s * zrows, zrows)],
                        acc.at[pl.ds(s * zrows, zrows)])
        plsc.subcore_barrier()
        wid = c * 16 + s
        for g in range(_GROUPS):
            w0 = wid * (_GROUPS * _GW) + g * _GW
            pltpu.sync_copy(i_hbm.at[pl.ds(w0, _GW)], idx_v)
            pltpu.sync_copy(m_hbm.at[pl.ds(w0 * 128, _GW * 128)], msg_v)
            for j in range(_GW):
                pltpu.sync_copy(msg_v.at[pl.ds(j * 128, 128)],
                                acc.at[idx_v.at[j]], add=True)
        plsc.subcore_barrier()

        @pl.when(s < 15)
        def _():
            pltpu.sync_copy(acc.at[pl.ds(s * 624, 624)],
                            o_hbm.at[c, pl.ds(s * 624, 624)])

        @pl.when(s == 15)
        def _():
            pltpu.sync_copy(acc.at[pl.ds(9360, 640)],
                            o_hbm.at[c, pl.ds(9360, 640)])

    return k(msg, idxw, zeros)


def _conv_mats(p):
    wnn = p['nn']['W']
    wall = wnn.reshape(_HID2, _HID, _HID2).transpose(1, 0, 2) \
              .reshape(_HID, _HID2 * _HID2)
    b2 = p['nn']['b'].reshape(_HID, _HID2)
    return wall, b2


def kernel(grid_graph_edge_index, grid_graph_edge_attr,
           ray_graph_edge_index, ray_graph_edge_attr, x, params):
    pad_e = _EP - _E
    ga = jnp.pad(grid_graph_edge_attr, ((0, pad_e), (0, 0)))
    ra = jnp.pad(ray_graph_edge_attr, ((0, pad_e), (0, 0)))
    src_g = jnp.pad(grid_graph_edge_index[0], (0, pad_e)).reshape(1, _EP)
    src_r = jnp.pad(ray_graph_edge_index[0], (0, pad_e)).reshape(1, _EP)
    dst_g = jnp.pad(grid_graph_edge_index[1], (0, pad_e),
                    constant_values=_N).reshape(_NWIN, 128)
    dst_r = jnp.pad(ray_graph_edge_index[1], (0, pad_e),
                    constant_values=_N).reshape(_NWIN, 128)

    rmat = jnp.repeat(jnp.eye(_HID2, dtype=_f32), _HID2, axis=1)
    smat = jnp.tile(jnp.eye(_HID2, dtype=_f32), (_HID2, 1))
    zeros = jnp.zeros((_NACC, _HID2), _f32)
    ones = jnp.ones((_EP, _HID2), _f32)

    x_grid_attr = _tc_mlp_rows(ga, params['encoder_grid'], _HID2, _BE)
    x_ray_attr = _tc_mlp_rows(ra, params['encoder_ray'], _HID2, _BE)
    h = _tc_mlp_rows(x, params['node_encoder'], _HID, _BN)
    cnt_g = _sc_scatter_add(ones, dst_g, zeros)
    cnt_r = _sc_scatter_add(ones, dst_r, zeros)

    for it in params['iters']:
        wall_g, b2_g = _conv_mats(it['conv_grid'])
        wall_r, b2_r = _conv_mats(it['conv_ray'])
        hsg = _sc_gather(h, src_g)
        hsr = _sc_gather(h, src_r)
        msg_g, msg_r = _tc_messages(hsg, hsr, x_ray_attr, x_grid_attr,
                                    wall_g, b2_g, wall_r, b2_r, rmat, smat)
        s_g = _sc_scatter_add(msg_g, dst_g, zeros)
        s_r = _sc_scatter_add(msg_r, dst_r, zeros)
        h = _tc_combine(s_g, s_r, cnt_g, cnt_r, h,
                        it['conv_grid']['root'], it['conv_grid']['bias'],
                        it['conv_ray']['root'], it['conv_ray']['bias'],
                        it['prep'])

    return _tc_mlp_rows(h, params['decoder'], 5, _BN)


# speedup vs baseline: 2.9825x; 1.0378x over previous
import functools

import jax
import jax.numpy as jnp
from jax import lax
from jax.experimental import pallas as pl
from jax.experimental.pallas import tpu as pltpu
from jax.experimental.pallas import tpu_sc as plsc

_N = 10000
_E = 160000
_HID = 32
_HID2 = 16
_EP = 163840
_NWIN = _EP // 128
_NACC = 10112
_BE = 2048
_BN = 2000

_f32 = jnp.float32


def _mlp(x, w1, b1, w2, b2, w3, b3, g, bt):
    h = jnp.maximum(jnp.dot(x, w1, preferred_element_type=_f32) + b1, 0.0)
    h = jnp.maximum(jnp.dot(h, w2, preferred_element_type=_f32) + b2, 0.0)
    y = jnp.dot(h, w3, preferred_element_type=_f32) + b3
    mu = jnp.mean(y, axis=-1, keepdims=True)
    var = jnp.mean((y - mu) ** 2, axis=-1, keepdims=True)
    return (y - mu) * lax.rsqrt(var + 1e-5) * g + bt


def _full_spec(a):
    nd = a.ndim
    return pl.BlockSpec(a.shape, lambda i, _nd=nd: (0,) * _nd)


def _mlp_rows_kernel(x_ref, w1, b1, w2, b2, w3, b3, g, bt, o_ref):
    o_ref[...] = _mlp(x_ref[...], w1[...], b1[...], w2[...], b2[...],
                      w3[...], b3[...], g[...], bt[...])


def _tc_mlp_rows(x, p, out_dim, block_rows):
    rows = x.shape[0]
    ws = (p['l1']['W'], p['l1']['b'], p['l2']['W'], p['l2']['b'],
          p['l3']['W'], p['l3']['b'], p['gamma'], p['beta'])
    return pl.pallas_call(
        _mlp_rows_kernel,
        grid=(rows // block_rows,),
        in_specs=[pl.BlockSpec((block_rows, x.shape[1]), lambda i: (i, 0))]
                 + [_full_spec(w) for w in ws],
        out_specs=pl.BlockSpec((block_rows, out_dim), lambda i: (i, 0)),
        out_shape=jax.ShapeDtypeStruct((rows, out_dim), _f32),
    )(x, *ws)


def _msg_kernel(hsg_ref, hsr_ref, ag_ref, ar_ref,
                wall_g, b2_g, wall_r, b2_r, rmat, smat,
                og_ref, or_ref):
    rm = rmat[...]
    sm = smat[...]

    def one(h_ref, a_ref, wall, b2):
        hs = h_ref[...]
        y = jnp.dot(hs, wall[...], preferred_element_type=_f32)
        a = jnp.dot(a_ref[...], rm, preferred_element_type=_f32)
        return (jnp.dot(y * a, sm, preferred_element_type=_f32)
                + jnp.dot(hs, b2[...], preferred_element_type=_f32))

    og_ref[...] = one(hsg_ref, ag_ref, wall_g, b2_g)
    or_ref[...] = one(hsr_ref, ar_ref, wall_r, b2_r)


def _tc_messages(hsg, hsr, attr_g, attr_r, wall_g, b2_g, wall_r, b2_r, rmat, smat):
    ws = (wall_g, b2_g, wall_r, b2_r, rmat, smat)
    return pl.pallas_call(
        _msg_kernel,
        grid=(_EP // _BE,),
        in_specs=[pl.BlockSpec((_BE, _HID), lambda i: (i, 0)),
                  pl.BlockSpec((_BE, _HID), lambda i: (i, 0)),
                  pl.BlockSpec((_BE, _HID2), lambda i: (i, 0)),
                  pl.BlockSpec((_BE, _HID2), lambda i: (i, 0))]
                 + [_full_spec(w) for w in ws],
        out_specs=[pl.BlockSpec((_BE, _HID2), lambda i: (i, 0)),
                   pl.BlockSpec((_BE, _HID2), lambda i: (i, 0))],
        out_shape=[jax.ShapeDtypeStruct((_EP, _HID2), _f32),
                   jax.ShapeDtypeStruct((_EP, _HID2), _f32)],
    )(hsg, hsr, attr_g, attr_r, *ws)


def _combine_kernel(sg0, sr0, cg0, cr0, h_ref,
                    root_g, bias_g, root_r, bias_r,
                    w1, b1, w2, b2, w3, b3, g, bt, o_ref):
    h = h_ref[...]
    cnt_g = jnp.maximum(cg0[...], 1.0)
    cnt_r = jnp.maximum(cr0[...], 1.0)
    xg = sg0[...] / cnt_g \
        + jnp.dot(h, root_g[...], preferred_element_type=_f32) + bias_g[...]
    xr = sr0[...] / cnt_r \
        + jnp.dot(h, root_r[...], preferred_element_type=_f32) + bias_r[...]
    cat = jnp.concatenate([xg, xr], axis=1)
    y = _mlp(cat, w1[...], b1[...], w2[...], b2[...], w3[...], b3[...],
             g[...], bt[...])
    o_ref[...] = y + h


def _tc_combine(sg, sr, cg, cr, h, root_g, bias_g, root_r, bias_r, prep):
    ws = (root_g, bias_g, root_r, bias_r,
          prep['l1']['W'], prep['l1']['b'], prep['l2']['W'], prep['l2']['b'],
          prep['l3']['W'], prep['l3']['b'], prep['gamma'], prep['beta'])
    part = pl.BlockSpec((_BN, _HID2), lambda i: (i, 0))
    return pl.pallas_call(
        _combine_kernel,
        grid=(_N // _BN,),
        in_specs=[part] * 4
                 + [pl.BlockSpec((_BN, _HID), lambda i: (i, 0))]
                 + [_full_spec(w) for w in ws],
        out_specs=pl.BlockSpec((_BN, _HID), lambda i: (i, 0)),
        out_shape=jax.ShapeDtypeStruct((_N, _HID), _f32),
    )(sg, sr, cg, cr, h, *ws)


def _sc_gather(table, idx2d):
    mesh = plsc.VectorSubcoreMesh(core_axis_name="c", subcore_axis_name="s")

    @functools.partial(
        pl.kernel, mesh=mesh,
        out_type=jax.ShapeDtypeStruct((_EP, _HID), _f32),
        compiler_params=pltpu.CompilerParams(use_tc_tiling_on_sc=False))
    def k(t_hbm, i_hbm, o_hbm):
        def body(i_vmem, o_vmem):
            pltpu.sync_copy(t_hbm.at[i_vmem.at[0]], o_vmem)

        pltpu.emit_pipeline(
            body,
            grid=(_NWIN,),
            in_specs=[pl.BlockSpec((1, 128), lambda i: (0, i))],
            out_specs=[pl.BlockSpec((128, _HID), lambda i: (i, 0))],
            core_axis_name=("c", "s"),
            dimension_semantics=(pltpu.PARALLEL,),
        )(i_hbm, o_hbm)

    return k(table, idx2d)


_GROUPS = 10
_GW = 8
_NBUF = 3


def _acc_zero(z_hbm, acc, s):
    zrows = _NACC // 16
    pltpu.sync_copy(z_hbm.at[pl.ds(s * zrows, zrows)],
                    acc.at[pl.ds(s * zrows, zrows)])


def _acc_out(acc, o_hbm, s):
    @pl.when(s < 15)
    def _():
        pltpu.sync_copy(acc.at[pl.ds(s * 624, 624)],
                        o_hbm.at[pl.ds(s * 624, 624)])

    @pl.when(s == 15)
    def _():
        pltpu.sync_copy(acc.at[pl.ds(9360, 640)], o_hbm.at[pl.ds(9360, 640)])


def _sc_scatter_add(msg_g, msg_r, idxw_g, idxw_r, zeros):
    mesh = plsc.VectorSubcoreMesh(core_axis_name="c", subcore_axis_name="s")

    @functools.partial(
        pl.kernel, mesh=mesh,
        out_type=(jax.ShapeDtypeStruct((_N, _HID2), _f32),
                  jax.ShapeDtypeStruct((_N, _HID2), _f32)),
        compiler_params=pltpu.CompilerParams(use_tc_tiling_on_sc=False),
        scratch_types=[pltpu.VMEM((_NBUF, _GW, 128), jnp.int32),
                       pltpu.VMEM((_NBUF, _GW * 128, _HID2), _f32),
                       pltpu.VMEM_SHARED((_NACC, _HID2), _f32),
                       pltpu.SemaphoreType.DMA,
                       pltpu.SemaphoreType.DMA,
                       pltpu.SemaphoreType.DMA])
    def k(mg_hbm, mr_hbm, ig_hbm, ir_hbm, z_hbm, og_hbm, or_hbm,
          idx_v, msg_v, acc, sem_i, sem_m, sem_s):
        c = lax.axis_index("c")
        s = lax.axis_index("s")
        _acc_zero(z_hbm, acc, s)
        plsc.subcore_barrier()

        def run(m_hbm, i_hbm, o_hbm):
            base = s * (_GROUPS * _GW)

            def start_loads(g):
                b = g % _NBUF
                w0 = base + g * _GW
                return (pltpu.async_copy(i_hbm.at[pl.ds(w0, _GW)],
                                         idx_v.at[b], sem_i),
                        pltpu.async_copy(m_hbm.at[pl.ds(w0 * 128, _GW * 128)],
                                        msg_v.at[b], sem_m))

            loads = {g: start_loads(g) for g in range(min(2, _GROUPS))}
            scats = {}
            for g in range(_GROUPS):
                for cp in loads.pop(g):
                    cp.wait()
                b = g % _NBUF
                scats[g] = [
                    pltpu.async_copy(msg_v.at[b, pl.ds(j * 128, 128)],
                                     acc.at[idx_v.at[b, j]], sem_s, add=True)
                    for j in range(_GW)]
                if g - 1 in scats:
                    for cp in scats.pop(g - 1):
                        cp.wait()
                if g + 2 < _GROUPS:
                    loads[g + 2] = start_loads(g + 2)
            for cps in scats.values():
                for cp in cps:
                    cp.wait()
            plsc.subcore_barrier()
            _acc_out(acc, o_hbm, s)

        @pl.when(c == 0)
        def _():
            run(mg_hbm, ig_hbm, og_hbm)

        @pl.when(c == 1)
        def _():
            run(mr_hbm, ir_hbm, or_hbm)

    return k(msg_g, msg_r, idxw_g, idxw_r, zeros)


def _sc_counts(idxw_g, idxw_r, zeros, ones128):
    mesh = plsc.VectorSubcoreMesh(core_axis_name="c", subcore_axis_name="s")

    @functools.partial(
        pl.kernel, mesh=mesh,
        out_type=(jax.ShapeDtypeStruct((_N, _HID2), _f32),
                  jax.ShapeDtypeStruct((_N, _HID2), _f32)),
        compiler_params=pltpu.CompilerParams(use_tc_tiling_on_sc=False),
        scratch_types=[pltpu.VMEM((_NBUF, _GW, 128), jnp.int32),
                       pltpu.VMEM((128, _HID2), _f32),
                       pltpu.VMEM_SHARED((_NACC, _HID2), _f32),
                       pltpu.SemaphoreType.DMA,
                       pltpu.SemaphoreType.DMA])
    def k(ig_hbm, ir_hbm, z_hbm, o1_hbm, og_hbm, or_hbm,
          idx_v, ones_v, acc, sem_i, sem_s):
        c = lax.axis_index("c")
        s = lax.axis_index("s")
        pltpu.sync_copy(o1_hbm, ones_v)
        _acc_zero(z_hbm, acc, s)
        plsc.subcore_barrier()

        def run(i_hbm, o_hbm):
            base = s * (_GROUPS * _GW)

            def start_load(g):
                b = g % _NBUF
                return pltpu.async_copy(i_hbm.at[pl.ds(base + g * _GW, _GW)],
                                        idx_v.at[b], sem_i)

            loads = {g: start_load(g) for g in range(min(2, _GROUPS))}
            scats = {}
            for g in range(_GROUPS):
                loads.pop(g).wait()
                b = g % _NBUF
                scats[g] = [
                    pltpu.async_copy(ones_v, acc.at[idx_v.at[b, j]],
                                     sem_s, add=True)
                    for j in range(_GW)]
                if g - 1 in scats:
                    for cp in scats.pop(g - 1):
                        cp.wait()
                if g + 2 < _GROUPS:
                    loads[g + 2] = start_load(g + 2)
            for cps in scats.values():
                for cp in cps:
                    cp.wait()
            plsc.subcore_barrier()
            _acc_out(acc, o_hbm, s)

        @pl.when(c == 0)
        def _():
            run(ig_hbm, og_hbm)

        @pl.when(c == 1)
        def _():
            run(ir_hbm, or_hbm)

    return k(idxw_g, idxw_r, zeros, ones128)


def _conv_mats(p):
    wnn = p['nn']['W']
    wall = wnn.reshape(_HID2, _HID, _HID2).transpose(1, 0, 2) \
              .reshape(_HID, _HID2 * _HID2)
    b2 = p['nn']['b'].reshape(_HID, _HID2)
    return wall, b2


def kernel(grid_graph_edge_index, grid_graph_edge_attr,
           ray_graph_edge_index, ray_graph_edge_attr, x, params):
    pad_e = _EP - _E
    ga = jnp.pad(grid_graph_edge_attr, ((0, pad_e), (0, 0)))
    ra = jnp.pad(ray_graph_edge_attr, ((0, pad_e), (0, 0)))
    src_g = jnp.pad(grid_graph_edge_index[0], (0, pad_e)).reshape(1, _EP)
    src_r = jnp.pad(ray_graph_edge_index[0], (0, pad_e)).reshape(1, _EP)
    dst_g = jnp.pad(grid_graph_edge_index[1], (0, pad_e),
                    constant_values=_N).reshape(_NWIN, 128)
    dst_r = jnp.pad(ray_graph_edge_index[1], (0, pad_e),
                    constant_values=_N).reshape(_NWIN, 128)

    rmat = jnp.repeat(jnp.eye(_HID2, dtype=_f32), _HID2, axis=1)
    smat = jnp.tile(jnp.eye(_HID2, dtype=_f32), (_HID2, 1))
    zeros = jnp.zeros((_NACC, _HID2), _f32)
    ones128 = jnp.ones((128, _HID2), _f32)

    x_grid_attr = _tc_mlp_rows(ga, params['encoder_grid'], _HID2, _BE)
    x_ray_attr = _tc_mlp_rows(ra, params['encoder_ray'], _HID2, _BE)
    h = _tc_mlp_rows(x, params['node_encoder'], _HID, _BN)
    cnt_g, cnt_r = _sc_counts(dst_g, dst_r, zeros, ones128)

    for it in params['iters']:
        wall_g, b2_g = _conv_mats(it['conv_grid'])
        wall_r, b2_r = _conv_mats(it['conv_ray'])
        hsg = _sc_gather(h, src_g)
        hsr = _sc_gather(h, src_r)
        msg_g, msg_r = _tc_messages(hsg, hsr, x_ray_attr, x_grid_attr,
                                    wall_g, b2_g, wall_r, b2_r, rmat, smat)
        s_g, s_r = _sc_scatter_add(msg_g, msg_r, dst_g, dst_r, zeros)
        h = _tc_combine(s_g, s_r, cnt_g, cnt_r, h,
                        it['conv_grid']['root'], it['conv_grid']['bias'],
                        it['conv_ray']['root'], it['conv_ray']['bias'],
                        it['prep'])

    return _tc_mlp_rows(h, params['decoder'], 5, _BN)
